# baseline (speedup 1.0000x reference)
import functools

import jax
import jax.numpy as jnp
from jax import lax
from jax.experimental import pallas as pl
from jax.experimental.pallas import tpu as pltpu
from jax.experimental.pallas import tpu_sc as plsc

N_NODES = 10000
N_PAD = 10240
N_EDGES = 320000
D = 128

NC = 2
NS = 16
NW = NC * NS
E_PER_W = 10240
E_TOT = NW * E_PER_W
CHUNK = 128
N_CHUNKS = E_PER_W // CHUNK
N_STAGES = 2
STAGE_CHUNKS = N_CHUNKS // N_STAGES
NBUF = 2
N_OUTER = STAGE_CHUNKS // NBUF
ROWS_PER_TILE = N_PAD // NS
LPC = CHUNK // 16


def _matmul_body(x_ref, wt_ref, o_ref):
    o_ref[...] = jnp.dot(x_ref[...], wt_ref[...],
                         preferred_element_type=jnp.float32)


def _add_body(a_ref, b_ref, o_ref):
    o_ref[...] = a_ref[...] + b_ref[...]


def _bcast_lane(v16, lane):
    return jnp.broadcast_to(lax.slice_in_dim(v16, lane, lane + 1), (16,))


_sc_mesh = plsc.VectorSubcoreMesh(core_axis_name="c", subcore_axis_name="s")


@functools.partial(
    pl.kernel,
    mesh=_sc_mesh,
    out_type=jax.ShapeDtypeStruct((NC, N_PAD, D), jnp.float32),
    scratch_types=[
        pltpu.VMEM((STAGE_CHUNKS, CHUNK), jnp.int32),
        pltpu.VMEM((STAGE_CHUNKS, CHUNK), jnp.int32),
        pltpu.VMEM((STAGE_CHUNKS, CHUNK), jnp.float32),
        pltpu.VMEM((NBUF, CHUNK, D), jnp.float32),
        pltpu.VMEM_SHARED((N_PAD, D), jnp.float32),
        [pltpu.SemaphoreType.DMA] * NBUF,
        [pltpu.SemaphoreType.DMA] * NBUF,
    ],
)
def _sc_aggregate(h_hbm, rows_hbm, cols_hbm, vals_hbm, out_hbm,
                  idx_c, idx_r, vals_v, gbuf, acc_sh, sems_g, sems_s):
    c = lax.axis_index("c")
    s = lax.axis_index("s")
    wid = c * NS + s
    rslice = pl.ds(s * ROWS_PER_TILE, ROWS_PER_TILE)

    def zero_row(j, carry):
        for i in range(D // 16):
            gbuf[0, j, pl.ds(i * 16, 16)] = jnp.zeros((16,), jnp.float32)
        return carry

    lax.fori_loop(0, CHUNK, zero_row, 0)
    for r in range(ROWS_PER_TILE // CHUNK):
        pltpu.sync_copy(gbuf.at[0],
                        acc_sh.at[pl.ds(s * ROWS_PER_TILE + r * CHUNK, CHUNK)])
    plsc.subcore_barrier()

    def scale_chunk(q, k):
        for jg in range(LPC):
            vv = vals_v[q, pl.ds(jg * 16, 16)]
            for lane in range(16):
                sv = _bcast_lane(vv, lane)
                j = jg * 16 + lane
                for i in range(D // 16):
                    sl = (k, j, pl.ds(i * 16, 16))
                    gbuf[sl] = gbuf[sl] * sv

    def stage_body(st, carry):
        csl = pl.ds(st * STAGE_CHUNKS, STAGE_CHUNKS)
        pltpu.sync_copy(cols_hbm.at[wid, csl], idx_c)
        pltpu.sync_copy(rows_hbm.at[wid, csl], idx_r)
        pltpu.sync_copy(vals_hbm.at[wid, csl], vals_v)

        for k in range(NBUF):
            pltpu.async_copy(h_hbm.at[idx_c.at[k]], gbuf.at[k], sems_g[k])

        def outer_body(t, carry2):
            for k in range(NBUF):
                q = t * NBUF + k
                pltpu.make_async_copy(h_hbm.at[idx_c.at[q]], gbuf.at[k],
                                      sems_g[k]).wait()
                scale_chunk(q, k)
                pltpu.async_copy(gbuf.at[k], acc_sh.at[idx_r.at[q]],
                                 sems_s[k], add=True)
            for k in range(NBUF):
                q = (t + 1) * NBUF + k

                @pl.when(q < STAGE_CHUNKS)
                def _():
                    pltpu.make_async_copy(h_hbm.at[pl.ds(0, CHUNK)],
                                          gbuf.at[k], sems_s[k]).wait()
                    pltpu.async_copy(h_hbm.at[idx_c.at[q]], gbuf.at[k],
                                     sems_g[k])
            return carry2

        lax.fori_loop(0, N_OUTER, outer_body, 0)

        for k in range(NBUF):
            pltpu.make_async_copy(h_hbm.at[pl.ds(0, CHUNK)], gbuf.at[k],
                                  sems_s[k]).wait()
        return carry

    lax.fori_loop(0, N_STAGES, stage_body, 0)

    plsc.subcore_barrier()
    pltpu.sync_copy(acc_sh.at[rslice], out_hbm.at[c, rslice])


def kernel(x, edge_index, edge_values, W):
    rows = edge_index[0].astype(jnp.int32)
    cols = edge_index[1].astype(jnp.int32)
    npad = E_TOT - N_EDGES
    rows = jnp.concatenate(
        [rows, jnp.full((npad,), N_PAD - 1, jnp.int32)]
    ).reshape(NW, N_CHUNKS, CHUNK)
    cols = jnp.concatenate(
        [cols, jnp.zeros((npad,), jnp.int32)]).reshape(NW, N_CHUNKS, CHUNK)
    vals = jnp.concatenate(
        [edge_values, jnp.zeros((npad,), jnp.float32)]
    ).reshape(NW, N_CHUNKS, CHUNK)

    blk = N_NODES // 10
    h = pl.pallas_call(
        _matmul_body,
        grid=(10,),
        in_specs=[
            pl.BlockSpec((blk, D), lambda i: (i, 0)),
            pl.BlockSpec((D, D), lambda i: (0, 0)),
        ],
        out_specs=pl.BlockSpec((blk, D), lambda i: (i, 0)),
        out_shape=jax.ShapeDtypeStruct((N_NODES, D), jnp.float32),
    )(x, W.T)

    partial = _sc_aggregate(h, rows, cols, vals)

    out = pl.pallas_call(
        _add_body,
        grid=(10,),
        in_specs=[
            pl.BlockSpec((blk, D), lambda i: (i, 0)),
            pl.BlockSpec((blk, D), lambda i: (i, 0)),
        ],
        out_specs=pl.BlockSpec((blk, D), lambda i: (i, 0)),
        out_shape=jax.ShapeDtypeStruct((N_NODES, D), jnp.float32),
    )(partial[0], partial[1])
    return out


# speedup vs baseline: 1.0214x; 1.0214x over previous
import functools

import jax
import jax.numpy as jnp
from jax import lax
from jax.experimental import pallas as pl
from jax.experimental.pallas import tpu as pltpu
from jax.experimental.pallas import tpu_sc as plsc

N_NODES = 10000
N_PAD = 10240
N_EDGES = 320000
D = 128

NC = 2
NS = 16
NW = NC * NS
E_PER_W = 10240
E_TOT = NW * E_PER_W
CHUNK = 128
N_CHUNKS = E_PER_W // CHUNK
N_STAGES = 2
STAGE_CHUNKS = N_CHUNKS // N_STAGES
NBUF = 2
N_OUTER = STAGE_CHUNKS // NBUF
ROWS_PER_TILE = N_PAD // NS
LPC = CHUNK // 16


def _matmul_body(x_ref, wt_ref, o_ref):
    o_ref[...] = jnp.dot(x_ref[...], wt_ref[...],
                         preferred_element_type=jnp.float32)


def _add_body(a_ref, b_ref, o_ref):
    o_ref[...] = a_ref[...] + b_ref[...]


def _bcast_lane(v16, lane):
    return jnp.broadcast_to(lax.slice_in_dim(v16, lane, lane + 1), (16,))


_sc_mesh = plsc.VectorSubcoreMesh(core_axis_name="c", subcore_axis_name="s")


@functools.partial(
    pl.kernel,
    mesh=_sc_mesh,
    out_type=jax.ShapeDtypeStruct((NC, N_PAD, D), jnp.float32),
    scratch_types=[
        pltpu.VMEM((STAGE_CHUNKS, CHUNK), jnp.int32),
        pltpu.VMEM((STAGE_CHUNKS, CHUNK), jnp.int32),
        pltpu.VMEM((STAGE_CHUNKS, CHUNK), jnp.float32),
        pltpu.VMEM((NBUF, CHUNK, D), jnp.float32),
        pltpu.VMEM_SHARED((N_PAD, D), jnp.float32),
        [pltpu.SemaphoreType.DMA] * NBUF,
        [pltpu.SemaphoreType.DMA] * NBUF,
    ],
)
def _sc_aggregate(h_hbm, rows_hbm, cols_hbm, vals_hbm, out_hbm,
                  idx_c, idx_r, vals_v, gbuf, acc_sh, sems_g, sems_s):
    c = lax.axis_index("c")
    s = lax.axis_index("s")
    wid = c * NS + s
    rslice = pl.ds(s * ROWS_PER_TILE, ROWS_PER_TILE)

    def zero_row(j, carry):
        for i in range(D // 16):
            gbuf[0, j, pl.ds(i * 16, 16)] = jnp.zeros((16,), jnp.float32)
        return carry

    lax.fori_loop(0, CHUNK, zero_row, 0)
    for r in range(ROWS_PER_TILE // CHUNK):
        pltpu.sync_copy(gbuf.at[0],
                        acc_sh.at[pl.ds(s * ROWS_PER_TILE + r * CHUNK, CHUNK)])
    plsc.subcore_barrier()

    def scale_chunk(q, k):
        for jg in range(LPC):
            vv = vals_v[q, pl.ds(jg * 16, 16)]
            for lane in range(16):
                sv = _bcast_lane(vv, lane)
                j = jg * 16 + lane
                for i in range(D // 16):
                    sl = (k, j, pl.ds(i * 16, 16))
                    gbuf[sl] = gbuf[sl] * sv

    def stage_body(st, carry):
        csl = pl.ds(st * STAGE_CHUNKS, STAGE_CHUNKS)
        pltpu.sync_copy(cols_hbm.at[wid, csl], idx_c)
        pltpu.sync_copy(rows_hbm.at[wid, csl], idx_r)
        pltpu.sync_copy(vals_hbm.at[wid, csl], vals_v)

        for k in range(NBUF):
            pltpu.async_copy(h_hbm.at[idx_c.at[k]], gbuf.at[k], sems_g[k])

        def outer_body(t, carry2):
            for k in range(NBUF):
                q = t * NBUF + k
                pltpu.make_async_copy(h_hbm.at[idx_c.at[q]], gbuf.at[k],
                                      sems_g[k]).wait()
                scale_chunk(q, k)
                pltpu.async_copy(gbuf.at[k], acc_sh.at[idx_r.at[q]],
                                 sems_s[k], add=True)
            for k in range(NBUF):
                q = (t + 1) * NBUF + k

                @pl.when(q < STAGE_CHUNKS)
                def _():
                    pltpu.make_async_copy(h_hbm.at[pl.ds(0, CHUNK)],
                                          gbuf.at[k], sems_s[k]).wait()
                    pltpu.async_copy(h_hbm.at[idx_c.at[q]], gbuf.at[k],
                                     sems_g[k])
            return carry2

        lax.fori_loop(0, N_OUTER, outer_body, 0)

        for k in range(NBUF):
            pltpu.make_async_copy(h_hbm.at[pl.ds(0, CHUNK)], gbuf.at[k],
                                  sems_s[k]).wait()
        return carry

    lax.fori_loop(0, N_STAGES, stage_body, 0)

    plsc.subcore_barrier()
    pltpu.sync_copy(acc_sh.at[rslice], out_hbm.at[c, rslice])


def kernel(x, edge_index, edge_values, W):
    rows = edge_index[0].astype(jnp.int32)
    cols = edge_index[1].astype(jnp.int32)
    npad = E_TOT - N_EDGES
    pad_rows = N_NODES + jnp.arange(npad, dtype=jnp.int32) % (N_PAD - N_NODES)
    rows = jnp.concatenate([rows, pad_rows]).reshape(NW, N_CHUNKS, CHUNK)
    cols = jnp.concatenate(
        [cols, jnp.zeros((npad,), jnp.int32)]).reshape(NW, N_CHUNKS, CHUNK)
    vals = jnp.concatenate(
        [edge_values, jnp.zeros((npad,), jnp.float32)]
    ).reshape(NW, N_CHUNKS, CHUNK)

    blk = N_NODES // 10
    h = pl.pallas_call(
        _matmul_body,
        grid=(10,),
        in_specs=[
            pl.BlockSpec((blk, D), lambda i: (i, 0)),
            pl.BlockSpec((D, D), lambda i: (0, 0)),
        ],
        out_specs=pl.BlockSpec((blk, D), lambda i: (i, 0)),
        out_shape=jax.ShapeDtypeStruct((N_NODES, D), jnp.float32),
    )(x, W.T)

    partial = _sc_aggregate(h, rows, cols, vals)

    out = pl.pallas_call(
        _add_body,
        grid=(10,),
        in_specs=[
            pl.BlockSpec((blk, D), lambda i: (i, 0)),
            pl.BlockSpec((blk, D), lambda i: (i, 0)),
        ],
        out_specs=pl.BlockSpec((blk, D), lambda i: (i, 0)),
        out_shape=jax.ShapeDtypeStruct((N_NODES, D), jnp.float32),
    )(partial[0], partial[1])
    return out


# speedup vs baseline: 2.1280x; 2.0834x over previous
import functools

import jax
import jax.numpy as jnp
from jax import lax
from jax.experimental import pallas as pl
from jax.experimental.pallas import tpu as pltpu
from jax.experimental.pallas import tpu_sc as plsc

N_NODES = 10000
N_PAD = 10240
N_EDGES = 320000
D = 128

NC = 2
NS = 16
NW = NC * NS
E_PER_W = 10240
E_TOT = NW * E_PER_W
CHUNK = 128
N_CHUNKS = E_PER_W // CHUNK
N_STAGES = 2
STAGE_CHUNKS = N_CHUNKS // N_STAGES
NBUF = 2
N_OUTER = STAGE_CHUNKS // NBUF
ROWS_PER_TILE = N_PAD // NS
LPC = CHUNK // 16


def _matmul_body(x_ref, wt_ref, o_ref):
    o_ref[...] = jnp.dot(x_ref[...], wt_ref[...],
                         preferred_element_type=jnp.float32)


def _add_body(a_ref, b_ref, o_ref):
    o_ref[...] = a_ref[...] + b_ref[...]


def _bcast_lane(v16, lane):
    return jnp.broadcast_to(lax.slice_in_dim(v16, lane, lane + 1), (16,))


_sc_mesh = plsc.VectorSubcoreMesh(core_axis_name="c", subcore_axis_name="s")


@functools.partial(
    pl.kernel,
    mesh=_sc_mesh,
    out_type=jax.ShapeDtypeStruct((NC, N_PAD, D), jnp.float32),
    scratch_types=[
        pltpu.VMEM((STAGE_CHUNKS, CHUNK), jnp.int32),
        pltpu.VMEM((STAGE_CHUNKS, CHUNK), jnp.int32),
        pltpu.VMEM((STAGE_CHUNKS, CHUNK), jnp.float32),
        pltpu.VMEM((NBUF, CHUNK, D), jnp.float32),
        pltpu.VMEM_SHARED((N_PAD, D), jnp.float32),
        [pltpu.SemaphoreType.DMA] * NBUF,
        [pltpu.SemaphoreType.DMA] * NBUF,
    ],
)
def _sc_aggregate(h_hbm, rows_hbm, cols_hbm, vals_hbm, out_hbm,
                  idx_c, idx_r, vals_v, gbuf, acc_sh, sems_g, sems_s):
    c = lax.axis_index("c")
    s = lax.axis_index("s")
    wid = (1 - c) * NS + s
    rslice = pl.ds(s * ROWS_PER_TILE, ROWS_PER_TILE)

    def zero_row(j, carry):
        for i in range(D // 16):
            gbuf[0, j, pl.ds(i * 16, 16)] = jnp.zeros((16,), jnp.float32)
        return carry

    lax.fori_loop(0, CHUNK, zero_row, 0)
    for r in range(ROWS_PER_TILE // CHUNK):
        pltpu.sync_copy(gbuf.at[0],
                        acc_sh.at[pl.ds(s * ROWS_PER_TILE + r * CHUNK, CHUNK)])
    plsc.subcore_barrier()

    def scale_chunk(q, k):
        for jg in range(LPC):
            vv = vals_v[q, pl.ds(jg * 16, 16)]
            for lane in range(16):
                sv = _bcast_lane(vv, lane)
                j = jg * 16 + lane
                for i in range(D // 16):
                    sl = (k, j, pl.ds(i * 16, 16))
                    gbuf[sl] = gbuf[sl] * sv

    def stage_body(st, carry):
        csl = pl.ds(st * STAGE_CHUNKS, STAGE_CHUNKS)
        pltpu.sync_copy(cols_hbm.at[wid, csl], idx_c)
        pltpu.sync_copy(rows_hbm.at[wid, csl], idx_r)
        pltpu.sync_copy(vals_hbm.at[wid, csl], vals_v)

        for k in range(NBUF):
            pltpu.async_copy(h_hbm.at[idx_c.at[k]], gbuf.at[k], sems_g[k])

        def outer_body(t, carry2):
            for k in range(NBUF):
                q = t * NBUF + k
                pltpu.make_async_copy(h_hbm.at[idx_c.at[q]], gbuf.at[k],
                                      sems_g[k]).wait()
                scale_chunk(q, k)
                pltpu.async_copy(gbuf.at[k], acc_sh.at[idx_r.at[q]],
                                 sems_s[k], add=True)
            for k in range(NBUF):
                q = (t + 1) * NBUF + k

                @pl.when(q < STAGE_CHUNKS)
                def _():
                    pltpu.make_async_copy(h_hbm.at[pl.ds(0, CHUNK)],
                                          gbuf.at[k], sems_s[k]).wait()
                    pltpu.async_copy(h_hbm.at[idx_c.at[q]], gbuf.at[k],
                                     sems_g[k])
            return carry2

        lax.fori_loop(0, N_OUTER, outer_body, 0)

        for k in range(NBUF):
            pltpu.make_async_copy(h_hbm.at[pl.ds(0, CHUNK)], gbuf.at[k],
                                  sems_s[k]).wait()
        return carry

    lax.fori_loop(0, N_STAGES, stage_body, 0)

    plsc.subcore_barrier()
    pltpu.sync_copy(acc_sh.at[rslice], out_hbm.at[c, rslice])


def kernel(x, edge_index, edge_values, W):
    rows = edge_index[0].astype(jnp.int32)
    cols = edge_index[1].astype(jnp.int32)
    npad = E_TOT - N_EDGES
    pad_rows = N_NODES + jnp.arange(npad, dtype=jnp.int32) % (N_PAD - N_NODES)
    rows = jnp.concatenate([rows, pad_rows]).reshape(NW, N_CHUNKS, CHUNK)
    pad_cols = jnp.arange(npad, dtype=jnp.int32) % N_NODES
    cols = jnp.concatenate([cols, pad_cols]).reshape(NW, N_CHUNKS, CHUNK)
    vals = jnp.concatenate(
        [edge_values, jnp.zeros((npad,), jnp.float32)]
    ).reshape(NW, N_CHUNKS, CHUNK)

    blk = N_NODES // 10
    h = pl.pallas_call(
        _matmul_body,
        grid=(10,),
        in_specs=[
            pl.BlockSpec((blk, D), lambda i: (i, 0)),
            pl.BlockSpec((D, D), lambda i: (0, 0)),
        ],
        out_specs=pl.BlockSpec((blk, D), lambda i: (i, 0)),
        out_shape=jax.ShapeDtypeStruct((N_NODES, D), jnp.float32),
    )(x, W.T)

    partial = _sc_aggregate(h, rows, cols, vals)

    out = pl.pallas_call(
        _add_body,
        grid=(10,),
        in_specs=[
            pl.BlockSpec((blk, D), lambda i: (i, 0)),
            pl.BlockSpec((blk, D), lambda i: (i, 0)),
        ],
        out_specs=pl.BlockSpec((blk, D), lambda i: (i, 0)),
        out_shape=jax.ShapeDtypeStruct((N_NODES, D), jnp.float32),
    )(partial[0], partial[1])
    return out


# speedup vs baseline: 2.5726x; 1.2089x over previous
import functools

import jax
import jax.numpy as jnp
from jax import lax
from jax.experimental import pallas as pl
from jax.experimental.pallas import tpu as pltpu
from jax.experimental.pallas import tpu_sc as plsc

N_NODES = 10000
N_PAD = 10240
N_EDGES = 320000
D = 128

NC = 2
NS = 16
NW = NC * NS
E_PER_W = 10240
E_TOT = NW * E_PER_W
CHUNK = 64
N_CHUNKS = E_PER_W // CHUNK
NBUF = 4
N_OUTER = N_CHUNKS // NBUF
ROWS_PER_TILE = N_PAD // NS
LPC = CHUNK // 16


def _matmul_body(x_ref, wt_ref, o_ref):
    o_ref[...] = jnp.dot(x_ref[...], wt_ref[...],
                         preferred_element_type=jnp.float32)


def _add_body(a_ref, b_ref, o_ref):
    o_ref[...] = a_ref[0] + b_ref[0]


def _bcast_lane(v16, lane):
    return jnp.broadcast_to(lax.slice_in_dim(v16, lane, lane + 1), (16,))


_sc_mesh = plsc.VectorSubcoreMesh(core_axis_name="c", subcore_axis_name="s")


@functools.partial(
    pl.kernel,
    mesh=_sc_mesh,
    out_type=jax.ShapeDtypeStruct((NC, N_PAD, D), jnp.float32),
    scratch_types=[
        pltpu.VMEM((E_PER_W,), jnp.int32),
        pltpu.VMEM((NBUF, CHUNK), jnp.int32),
        pltpu.VMEM((NBUF, CHUNK), jnp.float32),
        pltpu.VMEM((NBUF, CHUNK, D), jnp.float32),
        pltpu.VMEM_SHARED((N_PAD, D), jnp.float32),
        [pltpu.SemaphoreType.DMA] * NBUF,
        [pltpu.SemaphoreType.DMA] * NBUF,
        [pltpu.SemaphoreType.DMA] * NBUF,
    ],
)
def _sc_aggregate(h_hbm, rows_hbm, cols_hbm, vals_hbm, out_hbm,
                  idx_c, rowr, valr, gbuf, acc_sh, sems_g, sems_s, sems_rv):
    c = lax.axis_index("c")
    s = lax.axis_index("s")
    wid = c * NS + s
    rslice = pl.ds(s * ROWS_PER_TILE, ROWS_PER_TILE)
    ebase = wid * E_PER_W

    def zero_row(j, carry):
        for i in range(D // 16):
            gbuf[0, j, pl.ds(i * 16, 16)] = jnp.zeros((16,), jnp.float32)
        return carry

    lax.fori_loop(0, CHUNK, zero_row, 0)
    for r in range(ROWS_PER_TILE // CHUNK):
        pltpu.sync_copy(gbuf.at[0],
                        acc_sh.at[pl.ds(s * ROWS_PER_TILE + r * CHUNK, CHUNK)])

    pltpu.sync_copy(cols_hbm.at[pl.ds(ebase, E_PER_W)], idx_c)
    plsc.subcore_barrier()

    def issue_chunk(q, k):
        pltpu.async_copy(rows_hbm.at[pl.ds(ebase + q * CHUNK, CHUNK)],
                         rowr.at[k], sems_rv[k])
        pltpu.async_copy(vals_hbm.at[pl.ds(ebase + q * CHUNK, CHUNK)],
                         valr.at[k], sems_rv[k])
        pltpu.async_copy(h_hbm.at[idx_c.at[pl.ds(q * CHUNK, CHUNK)]],
                         gbuf.at[k], sems_g[k])

    def wait_gather(k):
        pltpu.make_async_copy(h_hbm.at[pl.ds(0, CHUNK)], gbuf.at[k],
                              sems_g[k]).wait()

    def wait_rv(k):
        pltpu.make_async_copy(rows_hbm.at[pl.ds(0, CHUNK)], rowr.at[k],
                              sems_rv[k]).wait()
        pltpu.make_async_copy(vals_hbm.at[pl.ds(0, CHUNK)], valr.at[k],
                              sems_rv[k]).wait()

    def wait_scatter(k):
        pltpu.make_async_copy(h_hbm.at[pl.ds(0, CHUNK)], gbuf.at[k],
                              sems_s[k]).wait()

    def scale_chunk(k):
        for jg in range(LPC):
            vv = valr[k, pl.ds(jg * 16, 16)]
            for lane in range(16):
                sv = _bcast_lane(vv, lane)
                j = jg * 16 + lane
                for i in range(D // 16):
                    sl = (k, j, pl.ds(i * 16, 16))
                    gbuf[sl] = gbuf[sl] * sv

    for k in range(2):
        issue_chunk(k, k)

    def outer_body(t, carry):
        for k in range(NBUF):
            g = t * NBUF + k
            wait_gather(k)
            wait_rv(k)
            scale_chunk(k)
            pltpu.async_copy(gbuf.at[k], acc_sh.at[rowr.at[k]], sems_s[k],
                             add=True)
            qn = g + 2
            kr = (k + 2) % NBUF

            @pl.when(qn < N_CHUNKS)
            def _():
                @pl.when(qn >= NBUF)
                def _():
                    wait_scatter(kr)

                issue_chunk(qn, kr)
        return carry

    lax.fori_loop(0, N_OUTER, outer_body, 0)

    for k in range(NBUF):
        wait_scatter(k)

    plsc.subcore_barrier()
    pltpu.sync_copy(acc_sh.at[rslice], out_hbm.at[c, rslice])


def kernel(x, edge_index, edge_values, W):
    rows = edge_index[0].astype(jnp.int32)
    cols = edge_index[1].astype(jnp.int32)
    npad = E_TOT - N_EDGES
    pad_idx = jnp.arange(npad, dtype=jnp.int32)
    rows = jnp.concatenate([rows, N_NODES + pad_idx % (N_PAD - N_NODES)])
    cols = jnp.concatenate([cols, pad_idx % N_NODES])
    vals = jnp.concatenate([edge_values, jnp.zeros((npad,), jnp.float32)])

    blk = N_NODES // 10
    h = pl.pallas_call(
        _matmul_body,
        grid=(10,),
        in_specs=[
            pl.BlockSpec((blk, D), lambda i: (i, 0)),
            pl.BlockSpec((D, D), lambda i: (0, 0)),
        ],
        out_specs=pl.BlockSpec((blk, D), lambda i: (i, 0)),
        out_shape=jax.ShapeDtypeStruct((N_NODES, D), jnp.float32),
    )(x, W.T)

    partial = _sc_aggregate(h, rows, cols, vals)

    out = pl.pallas_call(
        _add_body,
        grid=(10,),
        in_specs=[
            pl.BlockSpec((1, blk, D), lambda i: (0, i, 0)),
            pl.BlockSpec((1, blk, D), lambda i: (1, i, 0)),
        ],
        out_specs=pl.BlockSpec((blk, D), lambda i: (i, 0)),
        out_shape=jax.ShapeDtypeStruct((N_NODES, D), jnp.float32),
    )(partial, partial)
    return out


# speedup vs baseline: 2.6079x; 1.0137x over previous
import functools

import jax
import jax.numpy as jnp
from jax import lax
from jax.experimental import pallas as pl
from jax.experimental.pallas import tpu as pltpu
from jax.experimental.pallas import tpu_sc as plsc

N_NODES = 10000
N_PAD = 10240
N_EDGES = 320000
D = 128

NC = 2
NS = 16
NW = NC * NS
E_PER_W = 10240
E_TOT = NW * E_PER_W
CHUNK = 64
N_CHUNKS = E_PER_W // CHUNK
NBUF = 4
N_OUTER = N_CHUNKS // NBUF
ROWS_PER_TILE = N_PAD // NS
LPC = CHUNK // 16


def _matmul_body(x_ref, wt_ref, o_ref):
    o_ref[...] = jnp.dot(x_ref[...], wt_ref[...],
                         preferred_element_type=jnp.float32)


def _add_body(a_ref, b_ref, o_ref):
    o_ref[...] = a_ref[0] + b_ref[0]


def _bcast_lane(v16, lane):
    return jnp.broadcast_to(lax.slice_in_dim(v16, lane, lane + 1), (16,))


_sc_mesh = plsc.VectorSubcoreMesh(core_axis_name="c", subcore_axis_name="s")


@functools.partial(
    pl.kernel,
    mesh=_sc_mesh,
    out_type=jax.ShapeDtypeStruct((NC, N_PAD, D), jnp.float32),
    scratch_types=[
        pltpu.VMEM((E_PER_W,), jnp.int32),
        [pltpu.VMEM((CHUNK,), jnp.int32)] * NBUF,
        [pltpu.VMEM((CHUNK,), jnp.float32)] * NBUF,
        pltpu.VMEM((NBUF, CHUNK, D), jnp.float32),
        pltpu.VMEM_SHARED((N_PAD, D), jnp.float32),
        [pltpu.SemaphoreType.DMA] * NBUF,
        [pltpu.SemaphoreType.DMA] * NBUF,
        [pltpu.SemaphoreType.DMA] * NBUF,
    ],
)
def _sc_aggregate(h_hbm, rows_hbm, cols_hbm, vals_hbm, out_hbm,
                  idx_c, rowr, valr, gbuf, acc_sh, sems_g, sems_s, sems_rv):
    c = lax.axis_index("c")
    s = lax.axis_index("s")
    wid = c * NS + s
    rslice = pl.ds(s * ROWS_PER_TILE, ROWS_PER_TILE)
    ebase = wid * E_PER_W

    def zero_row(j, carry):
        for i in range(D // 16):
            gbuf[0, j, pl.ds(i * 16, 16)] = jnp.zeros((16,), jnp.float32)
        return carry

    lax.fori_loop(0, CHUNK, zero_row, 0)
    for r in range(ROWS_PER_TILE // CHUNK):
        pltpu.sync_copy(gbuf.at[0],
                        acc_sh.at[pl.ds(s * ROWS_PER_TILE + r * CHUNK, CHUNK)])

    pltpu.sync_copy(cols_hbm.at[pl.ds(ebase, E_PER_W)], idx_c)
    plsc.subcore_barrier()

    def issue_chunk(q, k):
        pltpu.async_copy(rows_hbm.at[pl.ds(ebase + q * CHUNK, CHUNK)],
                         rowr[k], sems_rv[k])
        pltpu.async_copy(vals_hbm.at[pl.ds(ebase + q * CHUNK, CHUNK)],
                         valr[k], sems_rv[k])
        pltpu.async_copy(h_hbm.at[idx_c.at[pl.ds(q * CHUNK, CHUNK)]],
                         gbuf.at[k], sems_g[k])

    def wait_gather(k):
        pltpu.make_async_copy(h_hbm.at[pl.ds(0, CHUNK)], gbuf.at[k],
                              sems_g[k]).wait()

    def wait_rv(k):
        pltpu.make_async_copy(rows_hbm.at[pl.ds(0, CHUNK)], rowr[k],
                              sems_rv[k]).wait()
        pltpu.make_async_copy(vals_hbm.at[pl.ds(0, CHUNK)], valr[k],
                              sems_rv[k]).wait()

    def wait_scatter(k):
        pltpu.make_async_copy(h_hbm.at[pl.ds(0, CHUNK)], gbuf.at[k],
                              sems_s[k]).wait()

    def scale_chunk(k):
        for jg in range(LPC):
            vv = valr[k][pl.ds(jg * 16, 16)]
            for lane in range(16):
                sv = _bcast_lane(vv, lane)
                j = jg * 16 + lane
                for i in range(D // 16):
                    sl = (k, j, pl.ds(i * 16, 16))
                    gbuf[sl] = gbuf[sl] * sv

    for k in range(2):
        issue_chunk(k, k)

    def outer_body(t, carry):
        for k in range(NBUF):
            g = t * NBUF + k
            wait_gather(k)
            wait_rv(k)
            scale_chunk(k)
            pltpu.async_copy(gbuf.at[k], acc_sh.at[rowr[k]], sems_s[k],
                             add=True)
            qn = g + 2
            kr = (k + 2) % NBUF

            @pl.when(qn < N_CHUNKS)
            def _():
                @pl.when(qn >= NBUF)
                def _():
                    wait_scatter(kr)

                issue_chunk(qn, kr)
        return carry

    lax.fori_loop(0, N_OUTER, outer_body, 0)

    for k in range(NBUF):
        wait_scatter(k)

    plsc.subcore_barrier()
    pltpu.sync_copy(acc_sh.at[rslice], out_hbm.at[c, rslice])


def kernel(x, edge_index, edge_values, W):
    rows = edge_index[0].astype(jnp.int32)
    cols = edge_index[1].astype(jnp.int32)
    npad = E_TOT - N_EDGES
    pad_idx = jnp.arange(npad, dtype=jnp.int32)
    rows = jnp.concatenate([rows, N_NODES + pad_idx % (N_PAD - N_NODES)])
    cols = jnp.concatenate([cols, pad_idx % N_NODES])
    vals = jnp.concatenate([edge_values, jnp.zeros((npad,), jnp.float32)])

    blk = N_NODES // 10
    h = pl.pallas_call(
        _matmul_body,
        grid=(10,),
        in_specs=[
            pl.BlockSpec((blk, D), lambda i: (i, 0)),
            pl.BlockSpec((D, D), lambda i: (0, 0)),
        ],
        out_specs=pl.BlockSpec((blk, D), lambda i: (i, 0)),
        out_shape=jax.ShapeDtypeStruct((N_NODES, D), jnp.float32),
    )(x, W.T)

    partial = _sc_aggregate(h, rows, cols, vals)

    out = pl.pallas_call(
        _add_body,
        grid=(10,),
        in_specs=[
            pl.BlockSpec((1, blk, D), lambda i: (0, i, 0)),
            pl.BlockSpec((1, blk, D), lambda i: (1, i, 0)),
        ],
        out_specs=pl.BlockSpec((blk, D), lambda i: (i, 0)),
        out_shape=jax.ShapeDtypeStruct((N_NODES, D), jnp.float32),
    )(partial, partial)
    return out


# speedup vs baseline: 2.6692x; 1.0235x over previous
import functools

import jax
import jax.numpy as jnp
from jax import lax
from jax.experimental import pallas as pl
from jax.experimental.pallas import tpu as pltpu
from jax.experimental.pallas import tpu_sc as plsc

N_NODES = 10000
N_PAD = 10240
N_EDGES = 320000
D = 128

NC = 2
NS = 16
NW = NC * NS
E_PER_W = N_EDGES // NW
CHUNK = 64
MAIN_CHUNKS = 156
TAIL = E_PER_W - MAIN_CHUNKS * CHUNK
NBUF = 4
N_OUTER = MAIN_CHUNKS // NBUF
ROWS_PER_TILE = N_PAD // NS
LPC = CHUNK // 16


def _matmul_body(x_ref, wt_ref, o_ref):
    o_ref[...] = jnp.dot(x_ref[...], wt_ref[...],
                         preferred_element_type=jnp.float32)


def _add_body(a_ref, b_ref, o_ref):
    o_ref[...] = a_ref[0] + b_ref[0]


def _bcast_lane(v16, lane):
    return jnp.broadcast_to(lax.slice_in_dim(v16, lane, lane + 1), (16,))


_sc_mesh = plsc.VectorSubcoreMesh(core_axis_name="c", subcore_axis_name="s")


@functools.partial(
    pl.kernel,
    mesh=_sc_mesh,
    out_type=jax.ShapeDtypeStruct((NC, N_PAD, D), jnp.float32),
    scratch_types=[
        pltpu.VMEM((E_PER_W,), jnp.int32),
        [pltpu.VMEM((CHUNK,), jnp.int32)] * NBUF,
        [pltpu.VMEM((CHUNK,), jnp.float32)] * NBUF,
        pltpu.VMEM((TAIL,), jnp.int32),
        pltpu.VMEM((TAIL,), jnp.float32),
        pltpu.VMEM((NBUF, CHUNK, D), jnp.float32),
        pltpu.VMEM_SHARED((N_PAD, D), jnp.float32),
        [pltpu.SemaphoreType.DMA] * NBUF,
        [pltpu.SemaphoreType.DMA] * NBUF,
        [pltpu.SemaphoreType.DMA] * NBUF,
    ],
)
def _sc_aggregate(h_hbm, rows_hbm, cols_hbm, vals_hbm, out_hbm,
                  idx_c, rowr, valr, rowt, valt, gbuf, acc_sh,
                  sems_g, sems_s, sems_rv):
    c = lax.axis_index("c")
    s = lax.axis_index("s")
    wid = c * NS + s
    rslice = pl.ds(s * ROWS_PER_TILE, ROWS_PER_TILE)
    ebase = wid * E_PER_W

    def zero_row(j, carry):
        for i in range(D // 16):
            gbuf[0, j, pl.ds(i * 16, 16)] = jnp.zeros((16,), jnp.float32)
        return carry

    lax.fori_loop(0, CHUNK, zero_row, 0)
    for r in range(ROWS_PER_TILE // CHUNK):
        pltpu.sync_copy(gbuf.at[0],
                        acc_sh.at[pl.ds(s * ROWS_PER_TILE + r * CHUNK, CHUNK)])

    pltpu.sync_copy(cols_hbm.at[pl.ds(ebase, E_PER_W)], idx_c)
    plsc.subcore_barrier()

    def issue_chunk(q, k):
        pltpu.async_copy(rows_hbm.at[pl.ds(ebase + q * CHUNK, CHUNK)],
                         rowr[k], sems_rv[k])
        pltpu.async_copy(vals_hbm.at[pl.ds(ebase + q * CHUNK, CHUNK)],
                         valr[k], sems_rv[k])
        pltpu.async_copy(h_hbm.at[idx_c.at[pl.ds(q * CHUNK, CHUNK)]],
                         gbuf.at[k], sems_g[k])

    def wait_gather(k):
        pltpu.make_async_copy(h_hbm.at[pl.ds(0, CHUNK)], gbuf.at[k],
                              sems_g[k]).wait()

    def wait_rv(k):
        pltpu.make_async_copy(rows_hbm.at[pl.ds(0, CHUNK)], rowr[k],
                              sems_rv[k]).wait()
        pltpu.make_async_copy(vals_hbm.at[pl.ds(0, CHUNK)], valr[k],
                              sems_rv[k]).wait()

    def wait_scatter(k):
        pltpu.make_async_copy(h_hbm.at[pl.ds(0, CHUNK)], gbuf.at[k],
                              sems_s[k]).wait()

    def scale_chunk(k):
        for jg in range(LPC):
            vv = valr[k][pl.ds(jg * 16, 16)]
            for lane in range(16):
                sv = _bcast_lane(vv, lane)
                j = jg * 16 + lane
                for i in range(D // 16):
                    sl = (k, j, pl.ds(i * 16, 16))
                    gbuf[sl] = gbuf[sl] * sv

    for k in range(2):
        issue_chunk(k, k)

    def outer_body(t, carry):
        for k in range(NBUF):
            g = t * NBUF + k
            wait_gather(k)
            wait_rv(k)
            scale_chunk(k)
            pltpu.async_copy(gbuf.at[k], acc_sh.at[rowr[k]], sems_s[k],
                             add=True)
            qn = g + 2
            kr = (k + 2) % NBUF

            @pl.when(qn < MAIN_CHUNKS)
            def _():
                @pl.when(qn >= NBUF)
                def _():
                    wait_scatter(kr)

                issue_chunk(qn, kr)
        return carry

    lax.fori_loop(0, N_OUTER, outer_body, 0)

    tbase = ebase + MAIN_CHUNKS * CHUNK
    wait_scatter(0)
    pltpu.async_copy(rows_hbm.at[pl.ds(tbase, TAIL)], rowt, sems_rv[0])
    pltpu.async_copy(vals_hbm.at[pl.ds(tbase, TAIL)], valt, sems_rv[0])
    pltpu.async_copy(h_hbm.at[idx_c.at[pl.ds(MAIN_CHUNKS * CHUNK, TAIL)]],
                     gbuf.at[0, pl.ds(0, TAIL)], sems_g[0])
    pltpu.make_async_copy(h_hbm.at[pl.ds(0, TAIL)],
                          gbuf.at[0, pl.ds(0, TAIL)], sems_g[0]).wait()
    pltpu.make_async_copy(rows_hbm.at[pl.ds(0, TAIL)], rowt,
                          sems_rv[0]).wait()
    pltpu.make_async_copy(vals_hbm.at[pl.ds(0, TAIL)], valt,
                          sems_rv[0]).wait()
    vvt = valt[pl.ds(0, 16)]
    for lane in range(16):
        svt = _bcast_lane(vvt, lane)
        for i in range(D // 16):
            slt = (0, lane, pl.ds(i * 16, 16))
            gbuf[slt] = gbuf[slt] * svt
    pltpu.async_copy(gbuf.at[0, pl.ds(0, TAIL)], acc_sh.at[rowt], sems_s[0],
                     add=True)
    pltpu.make_async_copy(h_hbm.at[pl.ds(0, TAIL)],
                          gbuf.at[0, pl.ds(0, TAIL)], sems_s[0]).wait()

    for k in range(1, NBUF):
        wait_scatter(k)

    plsc.subcore_barrier()
    pltpu.sync_copy(acc_sh.at[rslice], out_hbm.at[c, rslice])


def kernel(x, edge_index, edge_values, W):
    rows = edge_index[0].astype(jnp.int32)
    cols = edge_index[1].astype(jnp.int32)

    blk = N_NODES // 10
    h = pl.pallas_call(
        _matmul_body,
        grid=(10,),
        in_specs=[
            pl.BlockSpec((blk, D), lambda i: (i, 0)),
            pl.BlockSpec((D, D), lambda i: (0, 0)),
        ],
        out_specs=pl.BlockSpec((blk, D), lambda i: (i, 0)),
        out_shape=jax.ShapeDtypeStruct((N_NODES, D), jnp.float32),
    )(x, W.T)

    partial = _sc_aggregate(h, rows, cols, edge_values)

    out = pl.pallas_call(
        _add_body,
        grid=(10,),
        in_specs=[
            pl.BlockSpec((1, blk, D), lambda i: (0, i, 0)),
            pl.BlockSpec((1, blk, D), lambda i: (1, i, 0)),
        ],
        out_specs=pl.BlockSpec((blk, D), lambda i: (i, 0)),
        out_shape=jax.ShapeDtypeStruct((N_NODES, D), jnp.float32),
    )(partial, partial)
    return out


# speedup vs baseline: 2.7937x; 1.0466x over previous
import functools

import jax
import jax.numpy as jnp
from jax import lax
from jax.experimental import pallas as pl
from jax.experimental.pallas import tpu as pltpu
from jax.experimental.pallas import tpu_sc as plsc

N_NODES = 10000
N_PAD = 10240
N_EDGES = 320000
D = 128

NC = 2
NS = 16
NW = NC * NS
E_PER_W = N_EDGES // NW
CHUNK = 64
MAIN_CHUNKS = 156
TAIL = E_PER_W - MAIN_CHUNKS * CHUNK
NBUF = 4
N_OUTER = MAIN_CHUNKS // NBUF
ROWS_PER_TILE = N_PAD // NS
LPC = CHUNK // 16


def _matmul_body(x_ref, wt_ref, o_ref):
    o_ref[...] = jnp.dot(x_ref[...], wt_ref[...],
                         preferred_element_type=jnp.float32)


def _add_body(a_ref, b_ref, o_ref):
    o_ref[...] = a_ref[0] + b_ref[0]


def _bcast_lane(v16, lane):
    return jnp.broadcast_to(lax.slice_in_dim(v16, lane, lane + 1), (16,))


_sc_mesh = plsc.VectorSubcoreMesh(core_axis_name="c", subcore_axis_name="s")


@functools.partial(
    pl.kernel,
    mesh=_sc_mesh,
    out_type=jax.ShapeDtypeStruct((NC, N_PAD, D), jnp.float32),
    scratch_types=[
        pltpu.VMEM((E_PER_W,), jnp.int32),
        [pltpu.VMEM((CHUNK,), jnp.int32)] * NBUF,
        [pltpu.VMEM((CHUNK,), jnp.float32)] * NBUF,
        pltpu.VMEM((TAIL,), jnp.int32),
        pltpu.VMEM((TAIL,), jnp.float32),
        pltpu.VMEM((NBUF, CHUNK, D), jnp.float32),
        pltpu.VMEM_SHARED((N_PAD, D), jnp.float32),
        [pltpu.SemaphoreType.DMA] * NBUF,
        [pltpu.SemaphoreType.DMA] * NBUF,
        [pltpu.SemaphoreType.DMA] * NBUF,
    ],
)
def _sc_aggregate(h_hbm, ei_hbm, vals_hbm, out_hbm,
                  idx_c, rowr, valr, rowt, valt, gbuf, acc_sh,
                  sems_g, sems_s, sems_rv):
    c = lax.axis_index("c")
    s = lax.axis_index("s")
    wid = c * NS + s
    rslice = pl.ds(s * ROWS_PER_TILE, ROWS_PER_TILE)
    ebase = wid * E_PER_W

    def zero_row(j, carry):
        for i in range(D // 16):
            gbuf[0, j, pl.ds(i * 16, 16)] = jnp.zeros((16,), jnp.float32)
        return carry

    lax.fori_loop(0, CHUNK, zero_row, 0)
    for r in range(ROWS_PER_TILE // CHUNK):
        pltpu.sync_copy(gbuf.at[0],
                        acc_sh.at[pl.ds(s * ROWS_PER_TILE + r * CHUNK, CHUNK)])

    pltpu.sync_copy(ei_hbm.at[pl.ds(N_EDGES + ebase, E_PER_W)], idx_c)
    plsc.subcore_barrier()

    def issue_chunk(q, k):
        pltpu.async_copy(ei_hbm.at[pl.ds(ebase + q * CHUNK, CHUNK)],
                         rowr[k], sems_rv[k])
        pltpu.async_copy(vals_hbm.at[pl.ds(ebase + q * CHUNK, CHUNK)],
                         valr[k], sems_rv[k])
        pltpu.async_copy(h_hbm.at[idx_c.at[pl.ds(q * CHUNK, CHUNK)]],
                         gbuf.at[k], sems_g[k])

    def wait_gather(k):
        pltpu.make_async_copy(h_hbm.at[pl.ds(0, CHUNK)], gbuf.at[k],
                              sems_g[k]).wait()

    def wait_rv(k):
        pltpu.make_async_copy(ei_hbm.at[pl.ds(0, CHUNK)], rowr[k],
                              sems_rv[k]).wait()
        pltpu.make_async_copy(vals_hbm.at[pl.ds(0, CHUNK)], valr[k],
                              sems_rv[k]).wait()

    def wait_scatter(k):
        pltpu.make_async_copy(h_hbm.at[pl.ds(0, CHUNK)], gbuf.at[k],
                              sems_s[k]).wait()

    def scale_chunk(k):
        for jg in range(LPC):
            vv = valr[k][pl.ds(jg * 16, 16)]
            for lane in range(16):
                sv = _bcast_lane(vv, lane)
                j = jg * 16 + lane
                for i in range(D // 16):
                    sl = (k, j, pl.ds(i * 16, 16))
                    gbuf[sl] = gbuf[sl] * sv

    for k in range(2):
        issue_chunk(k, k)

    def outer_body(t, carry):
        for k in range(NBUF):
            g = t * NBUF + k
            wait_gather(k)
            wait_rv(k)
            scale_chunk(k)
            pltpu.async_copy(gbuf.at[k], acc_sh.at[rowr[k]], sems_s[k],
                             add=True)
            qn = g + 2
            kr = (k + 2) % NBUF

            @pl.when(qn < MAIN_CHUNKS)
            def _():
                @pl.when(qn >= NBUF)
                def _():
                    wait_scatter(kr)

                issue_chunk(qn, kr)
        return carry

    lax.fori_loop(0, N_OUTER, outer_body, 0)

    tbase = ebase + MAIN_CHUNKS * CHUNK
    wait_scatter(0)
    pltpu.async_copy(ei_hbm.at[pl.ds(tbase, TAIL)], rowt, sems_rv[0])
    pltpu.async_copy(vals_hbm.at[pl.ds(tbase, TAIL)], valt, sems_rv[0])
    pltpu.async_copy(h_hbm.at[idx_c.at[pl.ds(MAIN_CHUNKS * CHUNK, TAIL)]],
                     gbuf.at[0, pl.ds(0, TAIL)], sems_g[0])
    pltpu.make_async_copy(h_hbm.at[pl.ds(0, TAIL)],
                          gbuf.at[0, pl.ds(0, TAIL)], sems_g[0]).wait()
    pltpu.make_async_copy(ei_hbm.at[pl.ds(0, TAIL)], rowt,
                          sems_rv[0]).wait()
    pltpu.make_async_copy(vals_hbm.at[pl.ds(0, TAIL)], valt,
                          sems_rv[0]).wait()
    vvt = valt[pl.ds(0, 16)]
    for lane in range(16):
        svt = _bcast_lane(vvt, lane)
        for i in range(D // 16):
            slt = (0, lane, pl.ds(i * 16, 16))
            gbuf[slt] = gbuf[slt] * svt
    pltpu.async_copy(gbuf.at[0, pl.ds(0, TAIL)], acc_sh.at[rowt], sems_s[0],
                     add=True)
    pltpu.make_async_copy(h_hbm.at[pl.ds(0, TAIL)],
                          gbuf.at[0, pl.ds(0, TAIL)], sems_s[0]).wait()

    for k in range(1, NBUF):
        wait_scatter(k)

    plsc.subcore_barrier()
    pltpu.sync_copy(acc_sh.at[rslice], out_hbm.at[c, rslice])


def kernel(x, edge_index, edge_values, W):
    ei_flat = edge_index.astype(jnp.int32).reshape(-1)

    blk = N_NODES // 10
    h = pl.pallas_call(
        _matmul_body,
        grid=(10,),
        in_specs=[
            pl.BlockSpec((blk, D), lambda i: (i, 0)),
            pl.BlockSpec((D, D), lambda i: (0, 0)),
        ],
        out_specs=pl.BlockSpec((blk, D), lambda i: (i, 0)),
        out_shape=jax.ShapeDtypeStruct((N_NODES, D), jnp.float32),
    )(x, W.T)

    partial = _sc_aggregate(h, ei_flat, edge_values)

    out = pl.pallas_call(
        _add_body,
        grid=(10,),
        in_specs=[
            pl.BlockSpec((1, blk, D), lambda i: (0, i, 0)),
            pl.BlockSpec((1, blk, D), lambda i: (1, i, 0)),
        ],
        out_specs=pl.BlockSpec((blk, D), lambda i: (i, 0)),
        out_shape=jax.ShapeDtypeStruct((N_NODES, D), jnp.float32),
    )(partial, partial)
    return out
